# baseline (speedup 1.0000x reference)
import functools

import jax
import jax.numpy as jnp
from jax import lax
from jax.experimental import pallas as pl
from jax.experimental.pallas import tpu as pltpu
from jax.experimental.pallas import tpu_sc as plsc

V = 1000000
H = 64
B = 4096
L = 200
P = 512
M = 32
EPS = 1e-12

ROWS = B * L
LPAD = 256
LG = 208
NGRP = LG // 16
CT = LPAD * M

_NC, _NS = 2, 16
_NW = _NC * _NS
_SEQ_PER_W = B // _NW
_NSLOT = 4

_RSQRT_MAGIC = 0x5F3759DF


def _ctab_body(p_ref, i_ref, o_ref):
    o_ref[...] = p_ref[...][:, None, :] + i_ref[...][None, :, :]


def _build_ctab(pos_table, item_table):
    out = pl.pallas_call(
        _ctab_body,
        in_specs=[
            pl.BlockSpec((LPAD, H), lambda: (0, 0)),
            pl.BlockSpec((M, H), lambda: (0, 0)),
        ],
        out_specs=pl.BlockSpec((LPAD, M, H), lambda: (0, 0, 0)),
        out_shape=jax.ShapeDtypeStruct((LPAD, M, H), jnp.float32),
    )(pos_table, item_table)
    return out.reshape(CT, H)


_PADBLK = 512


def _pad_body(a_ref, b_ref, oa_ref, ob_ref):
    zeros = jnp.zeros((_PADBLK, LPAD - L), jnp.int32)
    oa_ref[:, :L] = a_ref[...]
    oa_ref[:, L:] = zeros
    ob_ref[:, :L] = b_ref[...]
    ob_ref[:, L:] = zeros


def _pad_ids(input_ids, ipid):
    return pl.pallas_call(
        _pad_body,
        grid=(B // _PADBLK,),
        in_specs=[
            pl.BlockSpec((_PADBLK, L), lambda i: (i, 0)),
            pl.BlockSpec((_PADBLK, L), lambda i: (i, 0)),
        ],
        out_specs=[
            pl.BlockSpec((_PADBLK, LPAD), lambda i: (i, 0)),
            pl.BlockSpec((_PADBLK, LPAD), lambda i: (i, 0)),
        ],
        out_shape=[
            jax.ShapeDtypeStruct((B, LPAD), jnp.int32),
            jax.ShapeDtypeStruct((B, LPAD), jnp.int32),
        ],
    )(input_ids, ipid)


def _vec_rsqrt(x):
    i = plsc.bitcast(x, jnp.int32)
    i = _RSQRT_MAGIC - lax.shift_right_logical(i, 1)
    y = plsc.bitcast(i, jnp.float32)
    xh = x * 0.5
    for _ in range(3):
        y = y * (1.5 - xh * y * y)
    return y


def _sc_fused(ids256, ipid256, word_table, ctab, gamma, beta):
    mesh = plsc.VectorSubcoreMesh(core_axis_name="c", subcore_axis_name="s")

    scratch = dict(
        g_v=pltpu.VMEM((H,), jnp.float32),
        b_v=pltpu.VMEM((H,), jnp.float32),
    )
    for t in range(_NSLOT):
        scratch[f"ebuf{t}"] = pltpu.VMEM((LG, H), jnp.float32)
        scratch[f"idrow{t}"] = pltpu.VMEM((L,), jnp.int32)
        scratch[f"iprow{t}"] = pltpu.VMEM((LPAD,), jnp.int32)
        scratch[f"semg{t}"] = pltpu.SemaphoreType.DMA
        scratch[f"semo{t}"] = pltpu.SemaphoreType.DMA
        scratch[f"semf{t}"] = pltpu.SemaphoreType.DMA
    for t in range(2):
        scratch[f"cidx{t}"] = pltpu.VMEM((LG,), jnp.int32)
        scratch[f"pbuf{t}"] = pltpu.VMEM((L, H), jnp.float32)
        scratch[f"semp{t}"] = pltpu.SemaphoreType.DMA

    @functools.partial(
        pl.kernel,
        mesh=mesh,
        out_type=jax.ShapeDtypeStruct((ROWS, H), jnp.float32),
        scratch_types=list(scratch.values()),
        compiler_params=pltpu.CompilerParams(
            use_tc_tiling_on_sc=False, needs_layout_passes=False),
    )
    def k(ids_hbm, ipid_hbm, word_hbm, ctab_hbm, gamma_hbm, beta_hbm, out_hbm,
          g_v, b_v, *slot_refs):
        ebuf = [slot_refs[6 * t + 0] for t in range(_NSLOT)]
        idrow = [slot_refs[6 * t + 1] for t in range(_NSLOT)]
        iprow = [slot_refs[6 * t + 2] for t in range(_NSLOT)]
        semg = [slot_refs[6 * t + 3] for t in range(_NSLOT)]
        semo = [slot_refs[6 * t + 4] for t in range(_NSLOT)]
        semf = [slot_refs[6 * t + 5] for t in range(_NSLOT)]
        cidx = [slot_refs[6 * _NSLOT + 3 * t + 0] for t in range(2)]
        pbuf = [slot_refs[6 * _NSLOT + 3 * t + 1] for t in range(2)]
        semp = [slot_refs[6 * _NSLOT + 3 * t + 2] for t in range(2)]

        wid = lax.axis_index("s") * _NC + lax.axis_index("c")
        seq0 = wid * _SEQ_PER_W

        pltpu.sync_copy(gamma_hbm, g_v)
        pltpu.sync_copy(beta_hbm, b_v)

        iota16 = lax.iota(jnp.int32, 16)

        def prefetch(c, t):
            pltpu.async_copy(
                ids_hbm.at[seq0 + c, pl.ds(0, L)], idrow[t], semf[t])
            pltpu.async_copy(ipid_hbm.at[seq0 + c], iprow[t], semf[t])

        def sg1(c, t, p):
            @pl.when(c >= _NSLOT)
            def _():
                pltpu.make_async_copy(
                    ebuf[t].at[pl.ds(0, L)], out_hbm.at[pl.ds(0, L)], semo[t]).wait()

            pltpu.make_async_copy(
                ids_hbm.at[seq0 + c, pl.ds(0, L)], idrow[t], semf[t]).wait()
            pltpu.make_async_copy(
                ipid_hbm.at[seq0 + c], iprow[t], semf[t]).wait()

            def grp(g, carry):
                m = iprow[t][pl.ds(g * 16, 16)] & (M - 1)
                l = g * 16 + iota16
                cidx[p][pl.ds(g * 16, 16)] = l * M + m
                return carry

            lax.fori_loop(0, NGRP, grp, 0)
            pltpu.async_copy(
                word_hbm.at[idrow[t]], ebuf[t].at[pl.ds(0, L)], semg[t])
            pltpu.async_copy(
                ctab_hbm.at[cidx[p].at[pl.ds(0, L)]], pbuf[p], semp[p])

            @pl.when(c + 2 < _SEQ_PER_W)
            def _():
                prefetch(c + 2, (t + 2) % _NSLOT)

        def pr(c, t, p):
            pltpu.make_async_copy(
                word_hbm.at[idrow[t]], ebuf[t].at[pl.ds(0, L)], semg[t]).wait()
            pltpu.make_async_copy(
                ctab_hbm.at[cidx[p].at[pl.ds(0, L)]], pbuf[p], semp[p]).wait()
            eb = ebuf[t]
            pb = pbuf[p]
            g4 = [g_v[pl.ds(16 * q, 16)] for q in range(4)]
            b4 = [b_v[pl.ds(16 * q, 16)] for q in range(4)]

            def grp(g, carry):
                rows = g * 16 + iota16
                nacc = 4
                ssum = [jnp.zeros((16,), jnp.float32) for _ in range(nacc)]
                ssq = [jnp.zeros((16,), jnp.float32) for _ in range(nacc)]
                for j in range(H):
                    col = (iota16 + j) & (H - 1)
                    v = plsc.load_gather(eb, [rows, col]) + plsc.load_gather(
                        pb, [rows, col])
                    ssum[j % nacc] = ssum[j % nacc] + v
                    ssq[j % nacc] = ssq[j % nacc] + v * v
                while len(ssum) > 1:
                    ssum = [a + b for a, b in zip(ssum[::2], ssum[1::2])]
                    ssq = [a + b for a, b in zip(ssq[::2], ssq[1::2])]
                mean = ssum[0] * (1.0 / H)
                var = ssq[0] * (1.0 / H) - mean * mean
                rstd = _vec_rsqrt(var + EPS)
                for r in range(16):
                    lane = jnp.full((16,), r, jnp.int32)
                    mb = jnp.take_along_axis(mean, lane, axis=0)
                    rb = jnp.take_along_axis(rstd, lane, axis=0)
                    row = g * 16 + r
                    for kq in range(4):
                        e = eb[row, pl.ds(kq * 16, 16)] + pb[row, pl.ds(kq * 16, 16)]
                        o = (e - mb) * (rb * g4[kq]) + b4[kq]
                        eb[row, pl.ds(kq * 16, 16)] = o
                return carry

            lax.fori_loop(0, NGRP, grp, 0)
            pltpu.async_copy(
                eb.at[pl.ds(0, L)], out_hbm.at[pl.ds((seq0 + c) * L, L)], semo[t])

        prefetch(jnp.int32(0), 0)
        prefetch(jnp.int32(1), 1)
        sg1(jnp.int32(0), 0, 0)
        sg1(jnp.int32(1), 1, 1)

        def step(i, carry):
            for kk in range(_NSLOT):
                c = i * _NSLOT + kk
                pr(c, kk, kk % 2)

                @pl.when(c + 2 < _SEQ_PER_W)
                def _():
                    sg1(c + 2, (kk + 2) % _NSLOT, kk % 2)

            return carry

        lax.fori_loop(0, _SEQ_PER_W // _NSLOT, step, 0)

        for t in range(_NSLOT):
            pltpu.make_async_copy(
                ebuf[t].at[pl.ds(0, L)], out_hbm.at[pl.ds(0, L)], semo[t]).wait()

    return k(ids256, ipid256, word_table, ctab, gamma, beta)


def kernel(input_ids, item_position_ids, word_table, pos_table, item_table, gamma, beta):
    ids256, ipid256 = _pad_ids(input_ids.astype(jnp.int32),
                               item_position_ids.astype(jnp.int32))
    ctab = _build_ctab(pos_table[:LPAD], item_table)
    out = _sc_fused(ids256, ipid256, word_table, ctab, gamma, beta)
    return out.reshape(B, L, H)


# speedup vs baseline: 1.1156x; 1.1156x over previous
import functools

import jax
import jax.numpy as jnp
from jax import lax
from jax.experimental import pallas as pl
from jax.experimental.pallas import tpu as pltpu
from jax.experimental.pallas import tpu_sc as plsc

V = 1000000
H = 64
B = 4096
L = 200
P = 512
M = 32
EPS = 1e-12

ROWS = B * L
LPAD = 256
LG = 208
NGRP = LG // 16
CT = LPAD * M

_NC, _NS = 2, 16
_NW = _NC * _NS
_SEQ_PER_W = B // _NW
_NSLOT = 4

_RSQRT_MAGIC = 0x5F3759DF


def _ctab_body(p_ref, i_ref, o_ref):
    o_ref[...] = p_ref[...][:, None, :] + i_ref[...][None, :, :]


def _build_ctab(pos_table, item_table):
    out = pl.pallas_call(
        _ctab_body,
        in_specs=[
            pl.BlockSpec((LPAD, H), lambda: (0, 0)),
            pl.BlockSpec((M, H), lambda: (0, 0)),
        ],
        out_specs=pl.BlockSpec((LPAD, M, H), lambda: (0, 0, 0)),
        out_shape=jax.ShapeDtypeStruct((LPAD, M, H), jnp.float32),
    )(pos_table, item_table)
    return out.reshape(CT, H)


_PADBLK = 512


def _pad_body(a_ref, b_ref, oa_ref, ob_ref):
    zeros = jnp.zeros((_PADBLK, LPAD - L), jnp.int32)
    oa_ref[:, :L] = a_ref[...]
    oa_ref[:, L:] = zeros
    ob_ref[:, :L] = b_ref[...]
    ob_ref[:, L:] = zeros


def _pad_ids(input_ids, ipid):
    return pl.pallas_call(
        _pad_body,
        grid=(B // _PADBLK,),
        in_specs=[
            pl.BlockSpec((_PADBLK, L), lambda i: (i, 0)),
            pl.BlockSpec((_PADBLK, L), lambda i: (i, 0)),
        ],
        out_specs=[
            pl.BlockSpec((_PADBLK, LPAD), lambda i: (i, 0)),
            pl.BlockSpec((_PADBLK, LPAD), lambda i: (i, 0)),
        ],
        out_shape=[
            jax.ShapeDtypeStruct((B, LPAD), jnp.int32),
            jax.ShapeDtypeStruct((B, LPAD), jnp.int32),
        ],
    )(input_ids, ipid)


def _vec_rsqrt(x):
    i = plsc.bitcast(x, jnp.int32)
    i = _RSQRT_MAGIC - lax.shift_right_logical(i, 1)
    y = plsc.bitcast(i, jnp.float32)
    xh = x * 0.5
    for _ in range(3):
        y = y * (1.5 - xh * y * y)
    return y


def _sc_fused(ids256, ipid256, word_table, ctab, gamma, beta):
    mesh = plsc.VectorSubcoreMesh(core_axis_name="c", subcore_axis_name="s")

    scratch = dict(
        ids_v=pltpu.VMEM((_SEQ_PER_W, LPAD), jnp.int32),
        ipid_v=pltpu.VMEM((_SEQ_PER_W, LPAD), jnp.int32),
        g_v=pltpu.VMEM((H,), jnp.float32),
        b_v=pltpu.VMEM((H,), jnp.float32),
    )
    for t in range(_NSLOT):
        scratch[f"cidx{t}"] = pltpu.VMEM((LG,), jnp.int32)
        scratch[f"ebuf{t}"] = pltpu.VMEM((LG, H), jnp.float32)
        scratch[f"semg{t}"] = pltpu.SemaphoreType.DMA
        scratch[f"semo{t}"] = pltpu.SemaphoreType.DMA

    @functools.partial(
        pl.kernel,
        mesh=mesh,
        out_type=jax.ShapeDtypeStruct((ROWS, H), jnp.float32),
        scratch_types=list(scratch.values()),
        compiler_params=pltpu.CompilerParams(
            use_tc_tiling_on_sc=False, needs_layout_passes=False),
    )
    def k(ids_hbm, ipid_hbm, word_hbm, ctab_hbm, gamma_hbm, beta_hbm, out_hbm,
          ids_v, ipid_v, g_v, b_v, *slot_refs):
        cidx = [slot_refs[4 * t + 0] for t in range(_NSLOT)]
        ebuf = [slot_refs[4 * t + 1] for t in range(_NSLOT)]
        semg = [slot_refs[4 * t + 2] for t in range(_NSLOT)]
        semo = [slot_refs[4 * t + 3] for t in range(_NSLOT)]

        wid = lax.axis_index("s") * _NC + lax.axis_index("c")
        seq0 = wid * _SEQ_PER_W

        pltpu.sync_copy(ids_hbm.at[pl.ds(seq0, _SEQ_PER_W)], ids_v)
        pltpu.sync_copy(ipid_hbm.at[pl.ds(seq0, _SEQ_PER_W)], ipid_v)
        pltpu.sync_copy(gamma_hbm, g_v)
        pltpu.sync_copy(beta_hbm, b_v)

        iota16 = lax.iota(jnp.int32, 16)

        def sg1(c, t):
            @pl.when(c >= _NSLOT)
            def _():
                pltpu.make_async_copy(
                    ebuf[t].at[pl.ds(0, L)], out_hbm.at[pl.ds(0, L)], semo[t]).wait()

            def grp(g, carry):
                m = ipid_v[c, pl.ds(g * 16, 16)]
                l = g * 16 + iota16
                cidx[t][pl.ds(g * 16, 16)] = l * M + m
                return carry

            lax.fori_loop(0, NGRP, grp, 0)
            pltpu.async_copy(ctab_hbm.at[cidx[t]], ebuf[t], semg[t])

        def sg2(c, t):
            pltpu.make_async_copy(ctab_hbm.at[cidx[t]], ebuf[t], semg[t]).wait()
            pltpu.async_copy(
                word_hbm.at[ids_v.at[c, pl.ds(0, L)]], ebuf[t].at[pl.ds(0, L)],
                semg[t], add=True)

        def pr(c, t):
            pltpu.make_async_copy(
                word_hbm.at[ids_v.at[c, pl.ds(0, L)]], ebuf[t].at[pl.ds(0, L)],
                semg[t]).wait()
            eb = ebuf[t]
            g4 = [g_v[pl.ds(16 * q, 16)] for q in range(4)]
            b4 = [b_v[pl.ds(16 * q, 16)] for q in range(4)]

            def grp(g, carry):
                rows = g * 16 + iota16
                nacc = 8
                ssum = [jnp.zeros((16,), jnp.float32) for _ in range(nacc)]
                ssq = [jnp.zeros((16,), jnp.float32) for _ in range(nacc)]
                for j in range(H):
                    col = (iota16 + j) & (H - 1)
                    v = plsc.load_gather(eb, [rows, col])
                    ssum[j % nacc] = ssum[j % nacc] + v
                    ssq[j % nacc] = ssq[j % nacc] + v * v
                while len(ssum) > 1:
                    ssum = [a + b for a, b in zip(ssum[::2], ssum[1::2])]
                    ssq = [a + b for a, b in zip(ssq[::2], ssq[1::2])]
                mean = ssum[0] * (1.0 / H)
                var = ssq[0] * (1.0 / H) - mean * mean
                rstd = _vec_rsqrt(var + EPS)
                for r in range(16):
                    lane = jnp.full((16,), r, jnp.int32)
                    mb = jnp.take_along_axis(mean, lane, axis=0)
                    rb = jnp.take_along_axis(rstd, lane, axis=0)
                    row = g * 16 + r
                    for kq in range(4):
                        e = eb[row, pl.ds(kq * 16, 16)]
                        o = (e - mb) * (rb * g4[kq]) + b4[kq]
                        eb[row, pl.ds(kq * 16, 16)] = o
                return carry

            lax.fori_loop(0, NGRP, grp, 0)
            pltpu.async_copy(
                eb.at[pl.ds(0, L)], out_hbm.at[pl.ds((seq0 + c) * L, L)], semo[t])

        sg1(jnp.int32(0), 0)
        sg1(jnp.int32(1), 1)
        sg2(jnp.int32(0), 0)

        def step(i, carry):
            for kk in range(_NSLOT):
                c = i * _NSLOT + kk

                @pl.when(c + 1 < _SEQ_PER_W)
                def _():
                    sg2(c + 1, (kk + 1) % _NSLOT)

                @pl.when(c + 2 < _SEQ_PER_W)
                def _():
                    sg1(c + 2, (kk + 2) % _NSLOT)

                pr(c, kk)
            return carry

        lax.fori_loop(0, _SEQ_PER_W // _NSLOT, step, 0)

        for t in range(_NSLOT):
            pltpu.make_async_copy(
                ebuf[t].at[pl.ds(0, L)], out_hbm.at[pl.ds(0, L)], semo[t]).wait()

    return k(ids256, ipid256, word_table, ctab, gamma, beta)


def kernel(input_ids, item_position_ids, word_table, pos_table, item_table, gamma, beta):
    ids256, ipid256 = _pad_ids(input_ids.astype(jnp.int32),
                               item_position_ids.astype(jnp.int32))
    ctab = _build_ctab(pos_table[:LPAD], item_table)
    out = _sc_fused(ids256, ipid256, word_table, ctab, gamma, beta)
    return out.reshape(B, L, H)
